# baseline (speedup 1.0000x reference)
import jax
import jax.numpy as jnp
from jax.experimental import pallas as pl
from jax.experimental.pallas import tpu as pltpu

_B = 4096
_SD = 200 * 64
_BLK = 256


def _add_body(x_ref, pos_ref, out_ref):
    out_ref[...] = x_ref[...] + pos_ref[...]


def kernel(x, pos_table):
    B, S, D = x.shape
    xt = x.transpose(1, 2, 0).reshape(S * D, B)
    post = pos_table[:S].reshape(S * D, 1)
    out_t = pl.pallas_call(
        _add_body,
        grid=(S * D // _BLK,),
        in_specs=[
            pl.BlockSpec((_BLK, B), lambda i: (i, 0)),
            pl.BlockSpec((_BLK, 1), lambda i: (i, 0)),
        ],
        out_specs=pl.BlockSpec((_BLK, B), lambda i: (i, 0)),
        out_shape=jax.ShapeDtypeStruct((S * D, B), x.dtype),
    )(xt, post)
    return out_t.reshape(S, D, B).transpose(2, 0, 1)


# speedup vs baseline: 1.0100x; 1.0100x over previous
import jax
import jax.numpy as jnp
from jax.experimental import pallas as pl
from jax.experimental.pallas import tpu as pltpu

_B = 4096
_SD = 200 * 64
_BLK = 800


def _add_body(x_ref, pos_ref, out_ref):
    out_ref[...] = x_ref[...] + pos_ref[...]


def kernel(x, pos_table):
    B, S, D = x.shape
    xt = x.transpose(1, 2, 0).reshape(S * D, B)
    post = pos_table[:S].reshape(S * D, 1)
    out_t = pl.pallas_call(
        _add_body,
        grid=(S * D // _BLK,),
        in_specs=[
            pl.BlockSpec((_BLK, B), lambda i: (i, 0)),
            pl.BlockSpec((_BLK, 1), lambda i: (i, 0)),
        ],
        out_specs=pl.BlockSpec((_BLK, B), lambda i: (i, 0)),
        out_shape=jax.ShapeDtypeStruct((S * D, B), x.dtype),
    )(xt, post)
    return out_t.reshape(S, D, B).transpose(2, 0, 1)
